# Initial kernel scaffold; baseline (speedup 1.0000x reference)
import jax
import jax.numpy as jnp
from jax.experimental import pallas as pl


def kernel(h, domain_embeddings, efficiency, Wt, Wg, bg, gamma, beta, k):
    raise NotImplementedError("write your pallas kernel here")


# speedup vs baseline: 2.0603x; 2.0603x over previous
import jax
import jax.numpy as jnp
from jax.experimental import pallas as pl
from jax.experimental.pallas import tpu as pltpu

_D = 1024
_P = 48
_B = 4096
_K = 4

_M_TILE = 512
_ROUT_TILE = 512

_INTERPRET = False


def _routing_body(h_ref, dom_ref, eff_ref, idx_ref, acc_ref):
    i = pl.program_id(0)
    n = pl.num_programs(0)

    @pl.when(i == 0)
    def _init():
        acc_ref[...] = jnp.zeros_like(acc_ref)

    acc_ref[...] += jnp.sum(h_ref[...], axis=0, keepdims=True)

    @pl.when(i == n - 1)
    def _final():
        hm = acc_ref[...] / _B
        norm = jnp.sqrt(jnp.sum(hm * hm))
        hn = hm / jnp.maximum(norm, 1e-12)
        dom = dom_ref[...]
        dnorm = jnp.sqrt(jnp.sum(dom * dom, axis=1, keepdims=True))
        en = dom / jnp.maximum(dnorm, 1e-12)
        scores = jnp.sum(en * hn, axis=1, keepdims=True)
        scores = scores + 0.1 * jnp.tanh(eff_ref[...])
        iota = jax.lax.broadcasted_iota(jnp.int32, (_P, 1), 0)
        neg = jnp.float32(-jnp.inf)
        for t in range(_K):
            m = jnp.max(scores)
            j = jnp.min(jnp.where(scores == m, iota, _P))
            idx_ref[t] = j
            scores = jnp.where(iota == j, neg, scores)


def _routing(h, dom, eff2d):
    return pl.pallas_call(
        _routing_body,
        grid=(_B // _ROUT_TILE,),
        in_specs=[
            pl.BlockSpec((_ROUT_TILE, _D), lambda i: (i, 0)),
            pl.BlockSpec((_P, _D), lambda i: (0, 0)),
            pl.BlockSpec((_P, 1), lambda i: (0, 0)),
        ],
        out_specs=pl.BlockSpec(memory_space=pltpu.SMEM),
        out_shape=jax.ShapeDtypeStruct((_K,), jnp.int32),
        scratch_shapes=[pltpu.VMEM((1, _D), jnp.float32)],
        interpret=_INTERPRET,
    )(h, dom, eff2d)


def _chain_body(idx_ref, x_ref, wt_ref, wg_ref, bg_ref, g_ref, b_ref,
                out_ref, acc_ref):
    s = pl.program_id(1)

    @pl.when(s == 0)
    def _():
        acc_ref[...] = x_ref[...]

    x = acc_ref[...]
    wt = wt_ref[0]
    wg = wg_ref[0]
    z = jax.lax.dot_general(x, wg, (((1,), (1,)), ((), ())),
                            preferred_element_type=jnp.float32) + bg_ref[0]
    gate = jax.nn.sigmoid(z)
    t = jax.lax.dot_general(x, wt, (((1,), (1,)), ((), ())),
                            preferred_element_type=jnp.float32)
    tr = t * jax.nn.sigmoid(t)
    y = x * (1.0 - gate) + tr * gate
    mu = jnp.mean(y, axis=1, keepdims=True)
    yc = y - mu
    var = jnp.mean(yc * yc, axis=1, keepdims=True)
    o = yc / jnp.sqrt(var + 1e-5) * g_ref[0] + b_ref[0]
    acc_ref[...] = o

    @pl.when(s == _K - 1)
    def _():
        out_ref[...] = o


def _chain(idx, h, Wt, Wg, bg, gamma, beta):
    grid_spec = pltpu.PrefetchScalarGridSpec(
        num_scalar_prefetch=1,
        grid=(_B // _M_TILE, _K),
        in_specs=[
            pl.BlockSpec((_M_TILE, _D), lambda m, s, idx: (m, 0)),
            pl.BlockSpec((1, _D, _D), lambda m, s, idx: (idx[s], 0, 0)),
            pl.BlockSpec((1, _D, _D), lambda m, s, idx: (idx[s], 0, 0)),
            pl.BlockSpec((1, 1, _D), lambda m, s, idx: (idx[s], 0, 0)),
            pl.BlockSpec((1, 1, _D), lambda m, s, idx: (idx[s], 0, 0)),
            pl.BlockSpec((1, 1, _D), lambda m, s, idx: (idx[s], 0, 0)),
        ],
        out_specs=pl.BlockSpec((_M_TILE, _D), lambda m, s, idx: (m, 0)),
        scratch_shapes=[pltpu.VMEM((_M_TILE, _D), jnp.float32)],
    )
    return pl.pallas_call(
        _chain_body,
        grid_spec=grid_spec,
        out_shape=jax.ShapeDtypeStruct((_B, _D), jnp.float32),
        interpret=_INTERPRET,
    )(idx, h, Wt, Wg, bg.reshape(_P, 1, _D), gamma.reshape(_P, 1, _D),
      beta.reshape(_P, 1, _D))


def kernel(h, domain_embeddings, efficiency, Wt, Wg, bg, gamma, beta, k):
    eff2d = efficiency.reshape(_P, 1)
    idx = _routing(h, domain_embeddings, eff2d)
    out = _chain(idx, h, Wt, Wg, bg, gamma, beta)
    idx = idx + jnp.asarray(k, dtype=idx.dtype) * 0
    return out, idx
